# baseline (speedup 1.0000x reference)
import functools

import jax
import jax.numpy as jnp
from jax import lax
from jax.experimental import pallas as pl
from jax.experimental.pallas import tpu as pltpu
from jax.experimental.pallas import tpu_sc as plsc

B = 16384
TEXT_DIM = 256
SENT_DIM = 16
OUT_DIM = TEXT_DIM + SENT_DIM
L = 16

NUM_CORES = 2
NUM_SUBCORES = 16
NUM_WORKERS = NUM_CORES * NUM_SUBCORES
BPW = B // NUM_WORKERS


def _encode_body(text_hbm, ids_hbm, table_hbm, out_hbm,
                 idx_v, table_v, sent_v, tsem, ssem):
    wid = lax.axis_index("s") * NUM_CORES + lax.axis_index("c")
    base = wid * BPW

    text_cp = pltpu.make_async_copy(
        text_hbm.at[pl.ds(base, BPW)],
        out_hbm.at[pl.ds(base, BPW), pl.ds(0, TEXT_DIM)], tsem)
    text_cp.start()

    pltpu.sync_copy(ids_hbm.at[pl.ds(base, BPW)], idx_v)
    pltpu.sync_copy(table_hbm, table_v)

    lane = lax.iota(jnp.int32, L)

    def lookup_group(p, _):
        rows = p * L + lane
        ids_vec = idx_v[pl.ds(p * L, L)]
        for j in range(SENT_DIM):
            col_j = jnp.full((L,), j, jnp.int32)
            vals = plsc.load_gather(table_v, [ids_vec, col_j])
            plsc.store_scatter(sent_v, [rows, col_j], vals)
        return 0

    lax.fori_loop(0, BPW // L, lookup_group, 0)

    sent_cp = pltpu.make_async_copy(
        sent_v, out_hbm.at[pl.ds(base, BPW), pl.ds(TEXT_DIM, SENT_DIM)], ssem)
    sent_cp.start()
    sent_cp.wait()
    text_cp.wait()


@functools.partial(jax.jit, static_argnames=())
def kernel(text_embed, sentiment_ids, sentiment_table):
    ids32 = sentiment_ids.astype(jnp.int32)
    mesh = plsc.VectorSubcoreMesh(core_axis_name="c", subcore_axis_name="s")
    enc = pl.kernel(
        _encode_body,
        mesh=mesh,
        compiler_params=pltpu.CompilerParams(needs_layout_passes=False),
        out_type=jax.ShapeDtypeStruct((B, OUT_DIM), jnp.float32),
        scratch_types=[
            pltpu.VMEM((BPW,), jnp.int32),
            pltpu.VMEM((3, SENT_DIM), jnp.float32),
            pltpu.VMEM((BPW, SENT_DIM), jnp.float32),
            pltpu.SemaphoreType.DMA,
            pltpu.SemaphoreType.DMA,
        ],
    )
    return enc(text_embed, ids32, sentiment_table)


# speedup vs baseline: 8.8890x; 8.8890x over previous
import functools

import jax
import jax.numpy as jnp
from jax import lax
from jax.experimental import pallas as pl
from jax.experimental.pallas import tpu as pltpu
from jax.experimental.pallas import tpu_sc as plsc

B = 16384
TEXT_DIM = 256
SENT_DIM = 16
OUT_DIM = TEXT_DIM + SENT_DIM
L = 16

NUM_CORES = 2
NUM_SUBCORES = 16
NUM_WORKERS = NUM_CORES * NUM_SUBCORES
BPW = B // NUM_WORKERS
NBUF = 4
CHUNK = 64
NCHUNK = BPW // CHUNK


def _encode_body(text_hbm, ids_hbm, table_hbm, out_hbm,
                 idx_v, table_v, *bufs_and_sems):
    bufs = bufs_and_sems[:NBUF]
    rsems = bufs_and_sems[NBUF:2 * NBUF]
    wsems = bufs_and_sems[2 * NBUF:3 * NBUF]

    wid = lax.axis_index("s") * NUM_CORES + lax.axis_index("c")
    base = wid * BPW

    pltpu.sync_copy(ids_hbm.at[pl.ds(base, BPW)], idx_v)
    pltpu.sync_copy(table_hbm, table_v)

    in_cp = [None] * NBUF
    out_cp = [None] * NBUF
    lane = lax.iota(jnp.int32, L)

    def start_read(r):
        br = r % NBUF
        in_cp[br] = pltpu.make_async_copy(
            text_hbm.at[pl.ds(base + r * CHUNK, CHUNK)],
            bufs[br].at[:, pl.ds(0, TEXT_DIM)], rsems[br])
        in_cp[br].start()

    for r in range(NBUF - 1):
        start_read(r)

    for c in range(NCHUNK):
        r = c + NBUF - 1
        if r < NCHUNK:
            br = r % NBUF
            if out_cp[br] is not None:
                out_cp[br].wait()
                out_cp[br] = None
            start_read(r)

        b = c % NBUF

        def lookup_group(p, _, _buf=bufs[b], _c=c):
            rows = p * L + lane
            ids_vec = idx_v[pl.ds(_c * CHUNK + p * L, L)]
            for j in range(SENT_DIM):
                col_j = jnp.full((L,), j, jnp.int32)
                vals = plsc.load_gather(table_v, [ids_vec, col_j])
                plsc.store_scatter(_buf, [rows, col_j + TEXT_DIM], vals)
            return 0

        lax.fori_loop(0, CHUNK // L, lookup_group, 0)

        in_cp[b].wait()
        out_cp[b] = pltpu.make_async_copy(
            bufs[b], out_hbm.at[pl.ds(base + c * CHUNK, CHUNK)], wsems[b])
        out_cp[b].start()

    for b in range(NBUF):
        if out_cp[b] is not None:
            out_cp[b].wait()


@functools.partial(jax.jit, static_argnames=())
def kernel(text_embed, sentiment_ids, sentiment_table):
    ids32 = sentiment_ids.astype(jnp.int32)
    mesh = plsc.VectorSubcoreMesh(core_axis_name="c", subcore_axis_name="s")
    enc = pl.kernel(
        _encode_body,
        mesh=mesh,
        compiler_params=pltpu.CompilerParams(needs_layout_passes=False),
        out_type=jax.ShapeDtypeStruct((B, OUT_DIM), jnp.float32),
        scratch_types=(
            [pltpu.VMEM((BPW,), jnp.int32),
             pltpu.VMEM((3, SENT_DIM), jnp.float32)]
            + [pltpu.VMEM((CHUNK, OUT_DIM), jnp.float32)] * NBUF
            + [pltpu.SemaphoreType.DMA] * (2 * NBUF)
        ),
    )
    return enc(text_embed, ids32, sentiment_table)


# speedup vs baseline: 8.8990x; 1.0011x over previous
import functools

import jax
import jax.numpy as jnp
from jax import lax
from jax.experimental import pallas as pl
from jax.experimental.pallas import tpu as pltpu
from jax.experimental.pallas import tpu_sc as plsc

B = 16384
TEXT_DIM = 256
SENT_DIM = 16
OUT_DIM = TEXT_DIM + SENT_DIM
L = 16

NUM_CORES = 2
NUM_SUBCORES = 16
NUM_WORKERS = NUM_CORES * NUM_SUBCORES
BPW = B // NUM_WORKERS
NBUF = 4
CHUNK = 64
NCHUNK = BPW // CHUNK


def _encode_body(text_hbm, ids_hbm, table_hbm, out_hbm,
                 idx_v, table_v, *bufs_and_sems):
    bufs = bufs_and_sems[:NBUF]
    rsems = bufs_and_sems[NBUF:2 * NBUF]
    wsems = bufs_and_sems[2 * NBUF:3 * NBUF]

    wid = lax.axis_index("s") * NUM_CORES + lax.axis_index("c")
    base = wid * BPW

    pltpu.sync_copy(ids_hbm.at[pl.ds(base, BPW)], idx_v)
    pltpu.sync_copy(table_hbm, table_v)

    in_cp = [None] * NBUF
    out_cp = [None] * NBUF
    lane = lax.iota(jnp.int32, L)

    def start_read(r):
        br = r % NBUF
        in_cp[br] = pltpu.make_async_copy(
            text_hbm.at[pl.ds(base + r * CHUNK, CHUNK)],
            bufs[br].at[:, pl.ds(0, TEXT_DIM)], rsems[br])
        in_cp[br].start()

    for r in range(NBUF - 1):
        start_read(r)

    for c in range(NCHUNK):
        r = c + NBUF - 1
        if r < NCHUNK:
            br = r % NBUF
            if out_cp[br] is not None:
                out_cp[br].wait()
                out_cp[br] = None
            start_read(r)

        b = c % NBUF

        def lookup_group(p, _, _buf=bufs[b], _c=c):
            rows = p * L + lane
            ids_vec = idx_v[pl.ds(_c * CHUNK + p * L, L)]
            for j in range(SENT_DIM):
                col_j = jnp.full((L,), j, jnp.int32)
                vals = plsc.load_gather(table_v, [ids_vec, col_j])
                plsc.store_scatter(_buf, [rows, col_j + TEXT_DIM], vals)
            return 0

        lax.fori_loop(0, CHUNK // L, lookup_group, 0)

        in_cp[b].wait()
        out_cp[b] = pltpu.make_async_copy(
            bufs[b], out_hbm.at[pl.ds(base + c * CHUNK, CHUNK)], wsems[b])
        out_cp[b].start()

    for b in range(NBUF):
        if out_cp[b] is not None:
            out_cp[b].wait()


@functools.partial(jax.jit, static_argnames=())
def kernel(text_embed, sentiment_ids, sentiment_table):
    ids32 = sentiment_ids.astype(jnp.int32)
    mesh = plsc.VectorSubcoreMesh(core_axis_name="c", subcore_axis_name="s")
    enc = pl.kernel(
        _encode_body,
        mesh=mesh,
        compiler_params=pltpu.CompilerParams(
            needs_layout_passes=False, use_tc_tiling_on_sc=True),
        out_type=jax.ShapeDtypeStruct((B, OUT_DIM), jnp.float32),
        scratch_types=(
            [pltpu.VMEM((BPW,), jnp.int32),
             pltpu.VMEM((3, SENT_DIM), jnp.float32)]
            + [pltpu.VMEM((CHUNK, OUT_DIM), jnp.float32)] * NBUF
            + [pltpu.SemaphoreType.DMA] * (2 * NBUF)
        ),
    )
    return enc(text_embed, ids32, sentiment_table)
